# baseline (speedup 1.0000x reference)
import functools

import jax
import jax.numpy as jnp
from jax import lax
from jax.experimental import pallas as pl
from jax.experimental.pallas import tpu as pltpu
from jax.experimental.pallas import tpu_sc as plsc

N = 10000
D = 128
DH = D // 2
E = 320000

NC = 2
NS = 16
NW = NC * NS

C = 128
K = 80
EPAD = NW * K * C
NPAD = 10240
ROWS_PER_TILE = NPAD // NS
NBUF = 4
DEGW = 8

_mesh = plsc.VectorSubcoreMesh(
    core_axis_name="c", subcore_axis_name="s", num_cores=NC, num_subcores=NS
)


@functools.partial(
    pl.kernel,
    out_type=jax.ShapeDtypeStruct((NC, NPAD, DEGW), jnp.float32),
    mesh=_mesh,
    compiler_params=pltpu.CompilerParams(use_tc_tiling_on_sc=False),
    scratch_types=[
        pltpu.VMEM((K, C), jnp.int32),
        pltpu.VMEM((C, DEGW), jnp.float32),
        pltpu.VMEM_SHARED((NPAD, DEGW), jnp.float32),
        pltpu.SemaphoreType.DMA,
    ],
)
def _sc_degree(dst_hbm, ones_hbm, zeros_hbm, degp_hbm, didx, ones_v, deg_sh, sem):
    c = lax.axis_index("c")
    s = lax.axis_index("s")
    w = c * NS + s
    pltpu.sync_copy(ones_hbm, ones_v)
    pltpu.sync_copy(dst_hbm.at[w], didx)
    pltpu.sync_copy(zeros_hbm, deg_sh.at[pl.ds(s * ROWS_PER_TILE, ROWS_PER_TILE)])
    plsc.subcore_barrier()

    @pl.loop(0, K)
    def _chunk(k):
        pltpu.async_copy(ones_v, deg_sh.at[didx.at[k]], sem, add=True)

    @pl.loop(0, K)
    def _drain(k):
        pltpu.make_async_copy(ones_v, deg_sh.at[didx.at[k]], sem).wait()

    plsc.subcore_barrier()
    pltpu.sync_copy(
        deg_sh.at[pl.ds(s * ROWS_PER_TILE, ROWS_PER_TILE)],
        degp_hbm.at[c, pl.ds(s * ROWS_PER_TILE, ROWS_PER_TILE)],
    )


@functools.partial(
    pl.kernel,
    out_type=[
        jax.ShapeDtypeStruct((NC, NPAD, DH), jnp.float32),
        jax.ShapeDtypeStruct((NC, NPAD, DH), jnp.float32),
    ],
    mesh=_mesh,
    compiler_params=pltpu.CompilerParams(use_tc_tiling_on_sc=False),
    scratch_types=[
        pltpu.VMEM((K, C), jnp.int32),
        pltpu.VMEM((K, C), jnp.int32),
        pltpu.VMEM((C, DH), jnp.float32),
        [pltpu.VMEM((C, DH), jnp.float32) for _ in range(NBUF)],
        pltpu.VMEM_SHARED((NPAD, DH), jnp.float32),
        [pltpu.SemaphoreType.DMA for _ in range(NBUF)],
        [pltpu.SemaphoreType.DMA for _ in range(NBUF)],
        pltpu.SemaphoreType.DMA,
    ],
)
def _sc_scatter(h_lo, h_hi, src_hbm, dst_hbm, out_lo, out_hi,
                sidx, didx, zbuf, bufs, acc_sh, sems, ssems, fsem):
    c = lax.axis_index("c")
    s = lax.axis_index("s")
    w = c * NS + s

    @pl.loop(0, C)
    def _zrow(r):
        for j in range(DH // 16):
            zbuf[r, pl.ds(j * 16, 16)] = jnp.zeros((16,), jnp.float32)

    pltpu.sync_copy(src_hbm.at[w], sidx)
    pltpu.sync_copy(dst_hbm.at[w], didx)

    stripe = pl.ds(s * ROWS_PER_TILE, ROWS_PER_TILE)
    for pi, (h_hbm, out_hbm) in enumerate(((h_lo, out_lo), (h_hi, out_hi))):
        for b in range(NBUF):
            pltpu.async_copy(h_hbm.at[sidx.at[b]], bufs[b], sems[b])
        if pi == 1:
            pltpu.make_async_copy(acc_sh.at[stripe], out_lo.at[c, stripe], fsem).wait()
        for i in range(ROWS_PER_TILE // C):
            pltpu.sync_copy(zbuf, acc_sh.at[pl.ds(s * ROWS_PER_TILE + i * C, C)])
        plsc.subcore_barrier()

        @pl.loop(0, K - NBUF, step=NBUF)
        def _group(k0):
            for b in range(NBUF):
                k = k0 + b
                pltpu.make_async_copy(h_hbm.at[sidx.at[k]], bufs[b], sems[b]).wait()
                pltpu.async_copy(bufs[b], acc_sh.at[didx.at[k]], ssems[b], add=True)
            for b in range(NBUF):
                k = k0 + b
                pltpu.make_async_copy(bufs[b], acc_sh.at[didx.at[k]], ssems[b]).wait()
                pltpu.async_copy(h_hbm.at[sidx.at[k + NBUF]], bufs[b], sems[b])

        for b in range(NBUF):
            k = K - NBUF + b
            pltpu.make_async_copy(h_hbm.at[sidx.at[k]], bufs[b], sems[b]).wait()
            pltpu.async_copy(bufs[b], acc_sh.at[didx.at[k]], ssems[b], add=True)
        for b in range(NBUF):
            k = K - NBUF + b
            pltpu.make_async_copy(bufs[b], acc_sh.at[didx.at[k]], ssems[b]).wait()

        plsc.subcore_barrier()
        if pi == 0:
            pltpu.async_copy(acc_sh.at[stripe], out_hbm.at[c, stripe], fsem)
        else:
            pltpu.sync_copy(acc_sh.at[stripe], out_hbm.at[c, stripe])


_B = 1000
_GRID = N // _B


def _stage1a_body(x_ref, w1_ref, h_ref):
    h_ref[...] = jnp.dot(x_ref[...], w1_ref[...], preferred_element_type=jnp.float32)


def _tc_stage1a(x, W1):
    return pl.pallas_call(
        _stage1a_body,
        grid=(_GRID,),
        in_specs=[
            pl.BlockSpec((_B, D), lambda i: (i, 0)),
            pl.BlockSpec((D, D), lambda i: (0, 0)),
        ],
        out_specs=pl.BlockSpec((_B, D), lambda i: (i, 0)),
        out_shape=jax.ShapeDtypeStruct((N, D), jnp.float32),
    )(x, W1)


def _stage1b_body(h_ref, degp_ref, lo_ref, hi_ref, dinv_ref):
    deg = degp_ref[0, :, 0:1] + degp_ref[1, :, 0:1] + 1.0
    dinv = lax.rsqrt(deg)
    hs = h_ref[...] * dinv
    lo_ref[...] = hs[:, :DH]
    hi_ref[...] = hs[:, DH:]
    dinv_ref[...] = dinv


def _tc_stage1(h1, degp):
    return pl.pallas_call(
        _stage1b_body,
        grid=(_GRID,),
        in_specs=[
            pl.BlockSpec((_B, D), lambda i: (i, 0)),
            pl.BlockSpec((NC, _B, DEGW), lambda i: (0, i, 0)),
        ],
        out_specs=[
            pl.BlockSpec((_B, DH), lambda i: (i, 0)),
            pl.BlockSpec((_B, DH), lambda i: (i, 0)),
            pl.BlockSpec((_B, 1), lambda i: (i, 0)),
        ],
        out_shape=[
            jax.ShapeDtypeStruct((N, DH), jnp.float32),
            jax.ShapeDtypeStruct((N, DH), jnp.float32),
            jax.ShapeDtypeStruct((N, 1), jnp.float32),
        ],
    )(h1, degp)


def _stage2_body(alo_ref, ahi_ref, hlo_ref, hhi_ref, dinv_ref, b_ref, w_ref,
                 lo_ref, hi_ref):
    dinv = dinv_ref[...]
    a_lo = alo_ref[0] + alo_ref[1] + hlo_ref[...]
    a_hi = ahi_ref[0] + ahi_ref[1] + hhi_ref[...]
    a = jnp.concatenate([a_lo, a_hi], axis=1)
    z = jnp.maximum(dinv * a + b_ref[...], 0.0)
    hs = jnp.dot(z, w_ref[...], preferred_element_type=jnp.float32) * dinv
    lo_ref[...] = hs[:, :DH]
    hi_ref[...] = hs[:, DH:]


def _tc_stage2(alo, ahi, hlo, hhi, dinv, b, W):
    return pl.pallas_call(
        _stage2_body,
        grid=(_GRID,),
        in_specs=[
            pl.BlockSpec((NC, _B, DH), lambda i: (0, i, 0)),
            pl.BlockSpec((NC, _B, DH), lambda i: (0, i, 0)),
            pl.BlockSpec((_B, DH), lambda i: (i, 0)),
            pl.BlockSpec((_B, DH), lambda i: (i, 0)),
            pl.BlockSpec((_B, 1), lambda i: (i, 0)),
            pl.BlockSpec((1, D), lambda i: (0, 0)),
            pl.BlockSpec((D, D), lambda i: (0, 0)),
        ],
        out_specs=[
            pl.BlockSpec((_B, DH), lambda i: (i, 0)),
            pl.BlockSpec((_B, DH), lambda i: (i, 0)),
        ],
        out_shape=[
            jax.ShapeDtypeStruct((N, DH), jnp.float32),
            jax.ShapeDtypeStruct((N, DH), jnp.float32),
        ],
    )(alo, ahi, hlo, hhi, dinv, b, W)


def _stage3_body(alo_ref, ahi_ref, hlo_ref, hhi_ref, dinv_ref, b_ref,
                 wfc_ref, bfc_ref, out_ref):
    a_lo = alo_ref[0] + alo_ref[1] + hlo_ref[...]
    a_hi = ahi_ref[0] + ahi_ref[1] + hhi_ref[...]
    a = jnp.concatenate([a_lo, a_hi], axis=1)
    z = jnp.maximum(dinv_ref[...] * a + b_ref[...], 0.0)
    out_ref[...] = (
        jnp.dot(z, wfc_ref[...], preferred_element_type=jnp.float32) + bfc_ref[...]
    )


def _tc_stage3(alo, ahi, hlo, hhi, dinv, b, Wfc8, bfc8):
    return pl.pallas_call(
        _stage3_body,
        grid=(_GRID,),
        in_specs=[
            pl.BlockSpec((NC, _B, DH), lambda i: (0, i, 0)),
            pl.BlockSpec((NC, _B, DH), lambda i: (0, i, 0)),
            pl.BlockSpec((_B, DH), lambda i: (i, 0)),
            pl.BlockSpec((_B, DH), lambda i: (i, 0)),
            pl.BlockSpec((_B, 1), lambda i: (i, 0)),
            pl.BlockSpec((1, D), lambda i: (0, 0)),
            pl.BlockSpec((D, DEGW), lambda i: (0, 0)),
            pl.BlockSpec((1, DEGW), lambda i: (0, 0)),
        ],
        out_specs=pl.BlockSpec((_B, DEGW), lambda i: (i, 0)),
        out_shape=jax.ShapeDtypeStruct((N, DEGW), jnp.float32),
    )(alo, ahi, hlo, hhi, dinv, b, Wfc8, bfc8)


def kernel(x, edge_index, W1, b1, W2, b2, Wfc, bfc):
    src = edge_index[0]
    dst = edge_index[1]
    pad = EPAD - E
    ar = jnp.arange(pad, dtype=jnp.int32)
    src3 = jnp.concatenate([src, (ar * 997) % N]).reshape(NW, K, C)
    dst3 = jnp.concatenate([dst, N + (ar % (NPAD - N))]).reshape(NW, K, C)

    ones = jnp.ones((C, DEGW), jnp.float32)
    zeros = jnp.zeros((ROWS_PER_TILE, DEGW), jnp.float32)

    h1 = _tc_stage1a(x, W1)
    degp = _sc_degree(dst3, ones, zeros)
    h1lo, h1hi, dinv = _tc_stage1(h1, degp)
    a1lo, a1hi = _sc_scatter(h1lo, h1hi, src3, dst3)
    h2lo, h2hi = _tc_stage2(a1lo, a1hi, h1lo, h1hi, dinv, b1.reshape(1, D), W2)
    a2lo, a2hi = _sc_scatter(h2lo, h2hi, src3, dst3)
    Wfc8 = jnp.pad(Wfc, ((0, 0), (0, DEGW - 2)))
    bfc8 = jnp.pad(bfc, (0, DEGW - 2)).reshape(1, DEGW)
    out8 = _tc_stage3(a2lo, a2hi, h2lo, h2hi, dinv, b2.reshape(1, D), Wfc8, bfc8)
    return out8[:, :2]


# speedup vs baseline: 1.1076x; 1.1076x over previous
import functools

import jax
import jax.numpy as jnp
from jax import lax
from jax.experimental import pallas as pl
from jax.experimental.pallas import tpu as pltpu
from jax.experimental.pallas import tpu_sc as plsc

N = 10000
D = 128
DH = D // 2
E = 320000

NC = 2
NS = 16
NW = NC * NS

C = 128
K = 80
EPAD = NW * K * C
NPAD = 10240
ROWS_PER_TILE = NPAD // NS
NBUF = 4
DEGW = 8

_mesh = plsc.VectorSubcoreMesh(
    core_axis_name="c", subcore_axis_name="s", num_cores=NC, num_subcores=NS
)


@functools.partial(
    pl.kernel,
    out_type=jax.ShapeDtypeStruct((NC, NPAD, DEGW), jnp.float32),
    mesh=_mesh,
    compiler_params=pltpu.CompilerParams(use_tc_tiling_on_sc=False),
    scratch_types=[
        pltpu.VMEM((K, C), jnp.int32),
        pltpu.VMEM((C, DEGW), jnp.float32),
        pltpu.VMEM_SHARED((NPAD, DEGW), jnp.float32),
        pltpu.SemaphoreType.DMA,
    ],
)
def _sc_degree(dst_hbm, ones_hbm, zeros_hbm, degp_hbm, didx, ones_v, deg_sh, sem):
    c = lax.axis_index("c")
    s = lax.axis_index("s")
    w = c * NS + s
    pltpu.sync_copy(ones_hbm, ones_v)
    pltpu.sync_copy(dst_hbm.at[w], didx)
    pltpu.sync_copy(zeros_hbm, deg_sh.at[pl.ds(s * ROWS_PER_TILE, ROWS_PER_TILE)])
    plsc.subcore_barrier()

    @pl.loop(0, K)
    def _chunk(k):
        pltpu.async_copy(ones_v, deg_sh.at[didx.at[k]], sem, add=True)

    @pl.loop(0, K)
    def _drain(k):
        pltpu.make_async_copy(ones_v, deg_sh.at[didx.at[k]], sem).wait()

    plsc.subcore_barrier()
    pltpu.sync_copy(
        deg_sh.at[pl.ds(s * ROWS_PER_TILE, ROWS_PER_TILE)],
        degp_hbm.at[c, pl.ds(s * ROWS_PER_TILE, ROWS_PER_TILE)],
    )


@functools.partial(
    pl.kernel,
    out_type=[
        jax.ShapeDtypeStruct((NC, NPAD, DH), jnp.float32),
        jax.ShapeDtypeStruct((NC, NPAD, DH), jnp.float32),
    ],
    mesh=_mesh,
    compiler_params=pltpu.CompilerParams(use_tc_tiling_on_sc=False),
    scratch_types=[
        pltpu.VMEM((K, C), jnp.int32),
        pltpu.VMEM((K, C), jnp.int32),
        pltpu.VMEM((C, DH), jnp.float32),
        [pltpu.VMEM((C, DH), jnp.float32) for _ in range(NBUF)],
        pltpu.VMEM_SHARED((NPAD, DH), jnp.float32),
        [pltpu.SemaphoreType.DMA for _ in range(NBUF)],
        pltpu.SemaphoreType.DMA,
    ],
)
def _sc_scatter(h_lo, h_hi, src_hbm, dst_hbm, out_lo, out_hi,
                sidx, didx, zbuf, bufs, acc_sh, sems, fsem):
    c = lax.axis_index("c")
    s = lax.axis_index("s")
    w = c * NS + s

    @pl.loop(0, C)
    def _zrow(r):
        for j in range(DH // 16):
            zbuf[r, pl.ds(j * 16, 16)] = jnp.zeros((16,), jnp.float32)

    pltpu.sync_copy(src_hbm.at[w], sidx)
    pltpu.sync_copy(dst_hbm.at[w], didx)

    stripe = pl.ds(s * ROWS_PER_TILE, ROWS_PER_TILE)
    for pi, (h_hbm, out_hbm) in enumerate(((h_lo, out_lo), (h_hi, out_hi))):
        for b in range(NBUF):
            pltpu.async_copy(h_hbm.at[sidx.at[b]], bufs[b], sems[b])
        if pi == 1:
            pltpu.make_async_copy(acc_sh.at[stripe], out_lo.at[c, stripe], fsem).wait()
        for i in range(ROWS_PER_TILE // C):
            pltpu.sync_copy(zbuf, acc_sh.at[pl.ds(s * ROWS_PER_TILE + i * C, C)])
        plsc.subcore_barrier()

        @pl.loop(0, K - NBUF, step=NBUF)
        def _group(k0):
            for b in range(NBUF):
                k = k0 + b
                pltpu.make_async_copy(h_hbm.at[sidx.at[k]], bufs[b], sems[b]).wait()
                pltpu.sync_copy(bufs[b], acc_sh.at[didx.at[k]], add=True)
                pltpu.async_copy(h_hbm.at[sidx.at[k + NBUF]], bufs[b], sems[b])

        for b in range(NBUF):
            k = K - NBUF + b
            pltpu.make_async_copy(h_hbm.at[sidx.at[k]], bufs[b], sems[b]).wait()
            pltpu.sync_copy(bufs[b], acc_sh.at[didx.at[k]], add=True)

        plsc.subcore_barrier()
        if pi == 0:
            pltpu.async_copy(acc_sh.at[stripe], out_hbm.at[c, stripe], fsem)
        else:
            pltpu.sync_copy(acc_sh.at[stripe], out_hbm.at[c, stripe])


_B = 1000
_GRID = N // _B


def _stage1a_body(x_ref, w1_ref, h_ref):
    h_ref[...] = jnp.dot(x_ref[...], w1_ref[...], preferred_element_type=jnp.float32)


def _tc_stage1a(x, W1):
    return pl.pallas_call(
        _stage1a_body,
        grid=(_GRID,),
        in_specs=[
            pl.BlockSpec((_B, D), lambda i: (i, 0)),
            pl.BlockSpec((D, D), lambda i: (0, 0)),
        ],
        out_specs=pl.BlockSpec((_B, D), lambda i: (i, 0)),
        out_shape=jax.ShapeDtypeStruct((N, D), jnp.float32),
    )(x, W1)


def _stage1b_body(h_ref, degp_ref, lo_ref, hi_ref, dinv_ref):
    deg = degp_ref[0, :, 0:1] + degp_ref[1, :, 0:1] + 1.0
    dinv = lax.rsqrt(deg)
    hs = h_ref[...] * dinv
    lo_ref[...] = hs[:, :DH]
    hi_ref[...] = hs[:, DH:]
    dinv_ref[...] = dinv


def _tc_stage1(h1, degp):
    return pl.pallas_call(
        _stage1b_body,
        grid=(_GRID,),
        in_specs=[
            pl.BlockSpec((_B, D), lambda i: (i, 0)),
            pl.BlockSpec((NC, _B, DEGW), lambda i: (0, i, 0)),
        ],
        out_specs=[
            pl.BlockSpec((_B, DH), lambda i: (i, 0)),
            pl.BlockSpec((_B, DH), lambda i: (i, 0)),
            pl.BlockSpec((_B, 1), lambda i: (i, 0)),
        ],
        out_shape=[
            jax.ShapeDtypeStruct((N, DH), jnp.float32),
            jax.ShapeDtypeStruct((N, DH), jnp.float32),
            jax.ShapeDtypeStruct((N, 1), jnp.float32),
        ],
    )(h1, degp)


def _stage2_body(alo_ref, ahi_ref, hlo_ref, hhi_ref, dinv_ref, b_ref, w_ref,
                 lo_ref, hi_ref):
    dinv = dinv_ref[...]
    a_lo = alo_ref[0] + alo_ref[1] + hlo_ref[...]
    a_hi = ahi_ref[0] + ahi_ref[1] + hhi_ref[...]
    a = jnp.concatenate([a_lo, a_hi], axis=1)
    z = jnp.maximum(dinv * a + b_ref[...], 0.0)
    hs = jnp.dot(z, w_ref[...], preferred_element_type=jnp.float32) * dinv
    lo_ref[...] = hs[:, :DH]
    hi_ref[...] = hs[:, DH:]


def _tc_stage2(alo, ahi, hlo, hhi, dinv, b, W):
    return pl.pallas_call(
        _stage2_body,
        grid=(_GRID,),
        in_specs=[
            pl.BlockSpec((NC, _B, DH), lambda i: (0, i, 0)),
            pl.BlockSpec((NC, _B, DH), lambda i: (0, i, 0)),
            pl.BlockSpec((_B, DH), lambda i: (i, 0)),
            pl.BlockSpec((_B, DH), lambda i: (i, 0)),
            pl.BlockSpec((_B, 1), lambda i: (i, 0)),
            pl.BlockSpec((1, D), lambda i: (0, 0)),
            pl.BlockSpec((D, D), lambda i: (0, 0)),
        ],
        out_specs=[
            pl.BlockSpec((_B, DH), lambda i: (i, 0)),
            pl.BlockSpec((_B, DH), lambda i: (i, 0)),
        ],
        out_shape=[
            jax.ShapeDtypeStruct((N, DH), jnp.float32),
            jax.ShapeDtypeStruct((N, DH), jnp.float32),
        ],
    )(alo, ahi, hlo, hhi, dinv, b, W)


def _stage3_body(alo_ref, ahi_ref, hlo_ref, hhi_ref, dinv_ref, b_ref,
                 wfc_ref, bfc_ref, out_ref):
    a_lo = alo_ref[0] + alo_ref[1] + hlo_ref[...]
    a_hi = ahi_ref[0] + ahi_ref[1] + hhi_ref[...]
    a = jnp.concatenate([a_lo, a_hi], axis=1)
    z = jnp.maximum(dinv_ref[...] * a + b_ref[...], 0.0)
    out_ref[...] = (
        jnp.dot(z, wfc_ref[...], preferred_element_type=jnp.float32) + bfc_ref[...]
    )


def _tc_stage3(alo, ahi, hlo, hhi, dinv, b, Wfc8, bfc8):
    return pl.pallas_call(
        _stage3_body,
        grid=(_GRID,),
        in_specs=[
            pl.BlockSpec((NC, _B, DH), lambda i: (0, i, 0)),
            pl.BlockSpec((NC, _B, DH), lambda i: (0, i, 0)),
            pl.BlockSpec((_B, DH), lambda i: (i, 0)),
            pl.BlockSpec((_B, DH), lambda i: (i, 0)),
            pl.BlockSpec((_B, 1), lambda i: (i, 0)),
            pl.BlockSpec((1, D), lambda i: (0, 0)),
            pl.BlockSpec((D, DEGW), lambda i: (0, 0)),
            pl.BlockSpec((1, DEGW), lambda i: (0, 0)),
        ],
        out_specs=pl.BlockSpec((_B, DEGW), lambda i: (i, 0)),
        out_shape=jax.ShapeDtypeStruct((N, DEGW), jnp.float32),
    )(alo, ahi, hlo, hhi, dinv, b, Wfc8, bfc8)


def kernel(x, edge_index, W1, b1, W2, b2, Wfc, bfc):
    src = edge_index[0]
    dst = edge_index[1]
    pad = EPAD - E
    ar = jnp.arange(pad, dtype=jnp.int32)
    src3 = jnp.concatenate([src, (ar * 997) % N]).reshape(NW, K, C)
    dst3 = jnp.concatenate([dst, N + (ar % (NPAD - N))]).reshape(NW, K, C)

    ones = jnp.ones((C, DEGW), jnp.float32)
    zeros = jnp.zeros((ROWS_PER_TILE, DEGW), jnp.float32)

    h1 = _tc_stage1a(x, W1)
    degp = _sc_degree(dst3, ones, zeros)
    h1lo, h1hi, dinv = _tc_stage1(h1, degp)
    a1lo, a1hi = _sc_scatter(h1lo, h1hi, src3, dst3)
    h2lo, h2hi = _tc_stage2(a1lo, a1hi, h1lo, h1hi, dinv, b1.reshape(1, D), W2)
    a2lo, a2hi = _sc_scatter(h2lo, h2hi, src3, dst3)
    Wfc8 = jnp.pad(Wfc, ((0, 0), (0, DEGW - 2)))
    bfc8 = jnp.pad(bfc, (0, DEGW - 2)).reshape(1, DEGW)
    out8 = _tc_stage3(a2lo, a2hi, h2lo, h2hi, dinv, b2.reshape(1, D), Wfc8, bfc8)
    return out8[:, :2]


# speedup vs baseline: 1.1229x; 1.0138x over previous
import functools

import jax
import jax.numpy as jnp
from jax import lax
from jax.experimental import pallas as pl
from jax.experimental.pallas import tpu as pltpu
from jax.experimental.pallas import tpu_sc as plsc

N = 10000
D = 128
DH = D // 2
E = 320000

NC = 2
NS = 16
NW = NC * NS

C = 128
K = 80
EPAD = NW * K * C
NPAD = 10240
ROWS_PER_TILE = NPAD // NS
NBUF = 4
DEGW = 8

_mesh = plsc.VectorSubcoreMesh(
    core_axis_name="c", subcore_axis_name="s", num_cores=NC, num_subcores=NS
)


@functools.partial(
    pl.kernel,
    out_type=jax.ShapeDtypeStruct((NC, NPAD, DEGW), jnp.float32),
    mesh=_mesh,
    compiler_params=pltpu.CompilerParams(use_tc_tiling_on_sc=False),
    scratch_types=[
        pltpu.VMEM((K, C), jnp.int32),
        pltpu.VMEM((C, DEGW), jnp.float32),
        pltpu.VMEM_SHARED((NPAD, DEGW), jnp.float32),
        pltpu.SemaphoreType.DMA,
    ],
)
def _sc_degree(dst_hbm, ones_hbm, zeros_hbm, degp_hbm, didx, ones_v, deg_sh, sem):
    c = lax.axis_index("c")
    s = lax.axis_index("s")
    w = c * NS + s
    pltpu.sync_copy(ones_hbm, ones_v)
    pltpu.sync_copy(dst_hbm.at[w], didx)
    pltpu.sync_copy(zeros_hbm, deg_sh.at[pl.ds(s * ROWS_PER_TILE, ROWS_PER_TILE)])
    plsc.subcore_barrier()

    @pl.loop(0, K)
    def _chunk(k):
        pltpu.async_copy(ones_v, deg_sh.at[didx.at[k]], sem, add=True)

    @pl.loop(0, K)
    def _drain(k):
        pltpu.make_async_copy(ones_v, deg_sh.at[didx.at[k]], sem).wait()

    plsc.subcore_barrier()
    pltpu.sync_copy(
        deg_sh.at[pl.ds(s * ROWS_PER_TILE, ROWS_PER_TILE)],
        degp_hbm.at[c, pl.ds(s * ROWS_PER_TILE, ROWS_PER_TILE)],
    )


@functools.partial(
    pl.kernel,
    out_type=[
        jax.ShapeDtypeStruct((NC, NPAD, DH), jnp.float32),
        jax.ShapeDtypeStruct((NC, NPAD, DH), jnp.float32),
    ],
    mesh=_mesh,
    compiler_params=pltpu.CompilerParams(use_tc_tiling_on_sc=False),
    scratch_types=[
        pltpu.VMEM((K, C), jnp.int32),
        pltpu.VMEM((K, C), jnp.int32),
        pltpu.VMEM((C, DH), jnp.float32),
        [pltpu.VMEM((C, DH), jnp.float32) for _ in range(NBUF)],
        pltpu.VMEM_SHARED((NPAD, DH), jnp.float32),
        [pltpu.SemaphoreType.DMA for _ in range(NBUF)],
        pltpu.SemaphoreType.DMA,
    ],
)
def _sc_scatter(h_lo, h_hi, src_hbm, dst_hbm, out_lo, out_hi,
                sidx, didx, zbuf, bufs, acc_sh, sems, fsem):
    c = lax.axis_index("c")
    s = lax.axis_index("s")
    w = c * NS + s

    @pl.loop(0, C)
    def _zrow(r):
        for j in range(DH // 16):
            zbuf[r, pl.ds(j * 16, 16)] = jnp.zeros((16,), jnp.float32)

    pltpu.sync_copy(src_hbm.at[w], sidx)
    pltpu.sync_copy(dst_hbm.at[w], didx)

    stripe = pl.ds(s * ROWS_PER_TILE, ROWS_PER_TILE)
    for pi, (h_hbm, out_hbm) in enumerate(((h_lo, out_lo), (h_hi, out_hi))):
        for b in range(NBUF):
            pltpu.async_copy(h_hbm.at[sidx.at[b]], bufs[b], sems[b])
        if pi == 1:
            pltpu.make_async_copy(acc_sh.at[stripe], out_lo.at[c, stripe], fsem).wait()
        for i in range(ROWS_PER_TILE // C):
            pltpu.sync_copy(zbuf, acc_sh.at[pl.ds(s * ROWS_PER_TILE + i * C, C)])
        plsc.subcore_barrier()

        @pl.loop(0, K - NBUF, step=NBUF)
        def _group(k0):
            for b in range(NBUF):
                k = k0 + b
                pltpu.make_async_copy(h_hbm.at[sidx.at[k]], bufs[b], sems[b]).wait()
                pltpu.sync_copy(bufs[b], acc_sh.at[didx.at[k]], add=True)
                pltpu.async_copy(h_hbm.at[sidx.at[k + NBUF]], bufs[b], sems[b])

        for b in range(NBUF):
            k = K - NBUF + b
            pltpu.make_async_copy(h_hbm.at[sidx.at[k]], bufs[b], sems[b]).wait()
            pltpu.sync_copy(bufs[b], acc_sh.at[didx.at[k]], add=True)

        plsc.subcore_barrier()
        if pi == 0:
            pltpu.async_copy(acc_sh.at[stripe], out_hbm.at[c, stripe], fsem)
        else:
            pltpu.sync_copy(acc_sh.at[stripe], out_hbm.at[c, stripe])


_B = 2000
_GRID = N // _B


def _stage1a_body(x_ref, w1_ref, h_ref):
    h_ref[...] = jnp.dot(x_ref[...], w1_ref[...], preferred_element_type=jnp.float32)


def _tc_stage1a(x, W1):
    return pl.pallas_call(
        _stage1a_body,
        grid=(_GRID,),
        in_specs=[
            pl.BlockSpec((_B, D), lambda i: (i, 0)),
            pl.BlockSpec((D, D), lambda i: (0, 0)),
        ],
        out_specs=pl.BlockSpec((_B, D), lambda i: (i, 0)),
        out_shape=jax.ShapeDtypeStruct((N, D), jnp.float32),
    )(x, W1)


def _stage1b_body(h_ref, degp_ref, lo_ref, hi_ref, dinv_ref):
    deg = degp_ref[0, :, 0:1] + degp_ref[1, :, 0:1] + 1.0
    dinv = lax.rsqrt(deg)
    hs = h_ref[...] * dinv
    lo_ref[...] = hs[:, :DH]
    hi_ref[...] = hs[:, DH:]
    dinv_ref[...] = dinv


def _tc_stage1(h1, degp):
    return pl.pallas_call(
        _stage1b_body,
        grid=(_GRID,),
        in_specs=[
            pl.BlockSpec((_B, D), lambda i: (i, 0)),
            pl.BlockSpec((NC, _B, DEGW), lambda i: (0, i, 0)),
        ],
        out_specs=[
            pl.BlockSpec((_B, DH), lambda i: (i, 0)),
            pl.BlockSpec((_B, DH), lambda i: (i, 0)),
            pl.BlockSpec((_B, 1), lambda i: (i, 0)),
        ],
        out_shape=[
            jax.ShapeDtypeStruct((N, DH), jnp.float32),
            jax.ShapeDtypeStruct((N, DH), jnp.float32),
            jax.ShapeDtypeStruct((N, 1), jnp.float32),
        ],
    )(h1, degp)


def _stage2_body(alo_ref, ahi_ref, hlo_ref, hhi_ref, dinv_ref, b_ref, w_ref,
                 lo_ref, hi_ref):
    dinv = dinv_ref[...]
    a_lo = alo_ref[0] + alo_ref[1] + hlo_ref[...]
    a_hi = ahi_ref[0] + ahi_ref[1] + hhi_ref[...]
    a = jnp.concatenate([a_lo, a_hi], axis=1)
    z = jnp.maximum(dinv * a + b_ref[...], 0.0)
    hs = jnp.dot(z, w_ref[...], preferred_element_type=jnp.float32) * dinv
    lo_ref[...] = hs[:, :DH]
    hi_ref[...] = hs[:, DH:]


def _tc_stage2(alo, ahi, hlo, hhi, dinv, b, W):
    return pl.pallas_call(
        _stage2_body,
        grid=(_GRID,),
        in_specs=[
            pl.BlockSpec((NC, _B, DH), lambda i: (0, i, 0)),
            pl.BlockSpec((NC, _B, DH), lambda i: (0, i, 0)),
            pl.BlockSpec((_B, DH), lambda i: (i, 0)),
            pl.BlockSpec((_B, DH), lambda i: (i, 0)),
            pl.BlockSpec((_B, 1), lambda i: (i, 0)),
            pl.BlockSpec((1, D), lambda i: (0, 0)),
            pl.BlockSpec((D, D), lambda i: (0, 0)),
        ],
        out_specs=[
            pl.BlockSpec((_B, DH), lambda i: (i, 0)),
            pl.BlockSpec((_B, DH), lambda i: (i, 0)),
        ],
        out_shape=[
            jax.ShapeDtypeStruct((N, DH), jnp.float32),
            jax.ShapeDtypeStruct((N, DH), jnp.float32),
        ],
    )(alo, ahi, hlo, hhi, dinv, b, W)


def _stage3_body(alo_ref, ahi_ref, hlo_ref, hhi_ref, dinv_ref, b_ref,
                 wfc_ref, bfc_ref, out_ref):
    a_lo = alo_ref[0] + alo_ref[1] + hlo_ref[...]
    a_hi = ahi_ref[0] + ahi_ref[1] + hhi_ref[...]
    a = jnp.concatenate([a_lo, a_hi], axis=1)
    z = jnp.maximum(dinv_ref[...] * a + b_ref[...], 0.0)
    out_ref[...] = (
        jnp.dot(z, wfc_ref[...], preferred_element_type=jnp.float32) + bfc_ref[...]
    )


def _tc_stage3(alo, ahi, hlo, hhi, dinv, b, Wfc8, bfc8):
    return pl.pallas_call(
        _stage3_body,
        grid=(_GRID,),
        in_specs=[
            pl.BlockSpec((NC, _B, DH), lambda i: (0, i, 0)),
            pl.BlockSpec((NC, _B, DH), lambda i: (0, i, 0)),
            pl.BlockSpec((_B, DH), lambda i: (i, 0)),
            pl.BlockSpec((_B, DH), lambda i: (i, 0)),
            pl.BlockSpec((_B, 1), lambda i: (i, 0)),
            pl.BlockSpec((1, D), lambda i: (0, 0)),
            pl.BlockSpec((D, DEGW), lambda i: (0, 0)),
            pl.BlockSpec((1, DEGW), lambda i: (0, 0)),
        ],
        out_specs=pl.BlockSpec((_B, DEGW), lambda i: (i, 0)),
        out_shape=jax.ShapeDtypeStruct((N, DEGW), jnp.float32),
    )(alo, ahi, hlo, hhi, dinv, b, Wfc8, bfc8)


def kernel(x, edge_index, W1, b1, W2, b2, Wfc, bfc):
    src = edge_index[0]
    dst = edge_index[1]
    pad = EPAD - E
    ar = jnp.arange(pad, dtype=jnp.int32)
    src3 = jnp.concatenate([src, (ar * 997) % N]).reshape(NW, K, C)
    dst3 = jnp.concatenate([dst, N + (ar % (NPAD - N))]).reshape(NW, K, C)

    ones = jnp.ones((C, DEGW), jnp.float32)
    zeros = jnp.zeros((ROWS_PER_TILE, DEGW), jnp.float32)

    h1 = _tc_stage1a(x, W1)
    degp = _sc_degree(dst3, ones, zeros)
    h1lo, h1hi, dinv = _tc_stage1(h1, degp)
    a1lo, a1hi = _sc_scatter(h1lo, h1hi, src3, dst3)
    h2lo, h2hi = _tc_stage2(a1lo, a1hi, h1lo, h1hi, dinv, b1.reshape(1, D), W2)
    a2lo, a2hi = _sc_scatter(h2lo, h2hi, src3, dst3)
    Wfc8 = jnp.pad(Wfc, ((0, 0), (0, DEGW - 2)))
    bfc8 = jnp.pad(bfc, (0, DEGW - 2)).reshape(1, DEGW)
    out8 = _tc_stage3(a2lo, a2hi, h2lo, h2hi, dinv, b2.reshape(1, D), Wfc8, bfc8)
    return out8[:, :2]


# speedup vs baseline: 1.1254x; 1.0022x over previous
import functools

import jax
import jax.numpy as jnp
from jax import lax
from jax.experimental import pallas as pl
from jax.experimental.pallas import tpu as pltpu
from jax.experimental.pallas import tpu_sc as plsc

N = 10000
D = 128
DH = D // 2
E = 320000

NC = 2
NS = 16
NW = NC * NS

C = 128
K = 80
EPAD = NW * K * C
NPAD = 10240
ROWS_PER_TILE = NPAD // NS
NBUF = 4
DEGW = 8

_mesh = plsc.VectorSubcoreMesh(
    core_axis_name="c", subcore_axis_name="s", num_cores=NC, num_subcores=NS
)


@functools.partial(
    pl.kernel,
    out_type=jax.ShapeDtypeStruct((NC, NPAD, DEGW), jnp.float32),
    mesh=_mesh,
    compiler_params=pltpu.CompilerParams(use_tc_tiling_on_sc=False),
    scratch_types=[
        pltpu.VMEM((K, C), jnp.int32),
        pltpu.VMEM((C, DEGW), jnp.float32),
        pltpu.VMEM_SHARED((NPAD, DEGW), jnp.float32),
        pltpu.SemaphoreType.DMA,
    ],
)
def _sc_degree(dst_hbm, ones_hbm, zeros_hbm, degp_hbm, didx, ones_v, deg_sh, sem):
    c = lax.axis_index("c")
    s = lax.axis_index("s")
    w = c * NS + s
    pltpu.sync_copy(ones_hbm, ones_v)
    pltpu.sync_copy(dst_hbm.at[w], didx)
    pltpu.sync_copy(zeros_hbm, deg_sh.at[pl.ds(s * ROWS_PER_TILE, ROWS_PER_TILE)])
    plsc.subcore_barrier()

    @pl.loop(0, K)
    def _chunk(k):
        pltpu.async_copy(ones_v, deg_sh.at[didx.at[k]], sem, add=True)

    @pl.loop(0, K)
    def _drain(k):
        pltpu.make_async_copy(ones_v, deg_sh.at[didx.at[k]], sem).wait()

    plsc.subcore_barrier()
    pltpu.sync_copy(
        deg_sh.at[pl.ds(s * ROWS_PER_TILE, ROWS_PER_TILE)],
        degp_hbm.at[c, pl.ds(s * ROWS_PER_TILE, ROWS_PER_TILE)],
    )


@functools.partial(
    pl.kernel,
    out_type=[
        jax.ShapeDtypeStruct((NC, NPAD, DH), jnp.float32),
        jax.ShapeDtypeStruct((NC, NPAD, DH), jnp.float32),
    ],
    mesh=_mesh,
    compiler_params=pltpu.CompilerParams(use_tc_tiling_on_sc=False),
    scratch_types=[
        pltpu.VMEM((K, C), jnp.int32),
        pltpu.VMEM((K, C), jnp.int32),
        pltpu.VMEM((C, DH), jnp.float32),
        [pltpu.VMEM((C, DH), jnp.float32) for _ in range(NBUF)],
        pltpu.VMEM_SHARED((NPAD, DH), jnp.float32),
        [pltpu.SemaphoreType.DMA for _ in range(NBUF)],
        pltpu.SemaphoreType.DMA,
    ],
)
def _sc_scatter(h_lo, h_hi, src_hbm, dst_hbm, out_lo, out_hi,
                sidx, didx, zbuf, bufs, acc_sh, sems, fsem):
    c = lax.axis_index("c")
    s = lax.axis_index("s")
    w = c * NS + s

    @pl.loop(0, C)
    def _zrow(r):
        for j in range(DH // 16):
            zbuf[r, pl.ds(j * 16, 16)] = jnp.zeros((16,), jnp.float32)

    pltpu.sync_copy(src_hbm.at[w], sidx)
    pltpu.sync_copy(dst_hbm.at[w], didx)

    stripe = pl.ds(s * ROWS_PER_TILE, ROWS_PER_TILE)
    for pi, (h_hbm, out_hbm) in enumerate(((h_lo, out_lo), (h_hi, out_hi))):
        for b in range(NBUF):
            pltpu.async_copy(h_hbm.at[sidx.at[b]], bufs[b], sems[b])
        if pi == 1:
            pltpu.make_async_copy(acc_sh.at[stripe], out_lo.at[c, stripe], fsem).wait()
        for i in range(ROWS_PER_TILE // C):
            pltpu.sync_copy(zbuf, acc_sh.at[pl.ds(s * ROWS_PER_TILE + i * C, C)])
        plsc.subcore_barrier()

        @pl.loop(0, K - NBUF, step=NBUF)
        def _group(k0):
            for b in range(NBUF):
                k = k0 + b
                pltpu.make_async_copy(h_hbm.at[sidx.at[k]], bufs[b], sems[b]).wait()
                pltpu.sync_copy(bufs[b], acc_sh.at[didx.at[k]], add=True)
                pltpu.async_copy(h_hbm.at[sidx.at[k + NBUF]], bufs[b], sems[b])

        for b in range(NBUF):
            k = K - NBUF + b
            pltpu.make_async_copy(h_hbm.at[sidx.at[k]], bufs[b], sems[b]).wait()
            pltpu.sync_copy(bufs[b], acc_sh.at[didx.at[k]], add=True)

        plsc.subcore_barrier()
        if pi == 0:
            pltpu.async_copy(acc_sh.at[stripe], out_hbm.at[c, stripe], fsem)
        else:
            pltpu.sync_copy(acc_sh.at[stripe], out_hbm.at[c, stripe])


_B = 5000
_GRID = N // _B


def _stage1a_body(x_ref, w1_ref, h_ref):
    h_ref[...] = jnp.dot(x_ref[...], w1_ref[...], preferred_element_type=jnp.float32)


def _tc_stage1a(x, W1):
    return pl.pallas_call(
        _stage1a_body,
        grid=(_GRID,),
        in_specs=[
            pl.BlockSpec((_B, D), lambda i: (i, 0)),
            pl.BlockSpec((D, D), lambda i: (0, 0)),
        ],
        out_specs=pl.BlockSpec((_B, D), lambda i: (i, 0)),
        out_shape=jax.ShapeDtypeStruct((N, D), jnp.float32),
    )(x, W1)


def _stage1b_body(h_ref, degp_ref, lo_ref, hi_ref, dinv_ref):
    deg = degp_ref[0, :, 0:1] + degp_ref[1, :, 0:1] + 1.0
    dinv = lax.rsqrt(deg)
    hs = h_ref[...] * dinv
    lo_ref[...] = hs[:, :DH]
    hi_ref[...] = hs[:, DH:]
    dinv_ref[...] = dinv


def _tc_stage1(h1, degp):
    return pl.pallas_call(
        _stage1b_body,
        grid=(_GRID,),
        in_specs=[
            pl.BlockSpec((_B, D), lambda i: (i, 0)),
            pl.BlockSpec((NC, _B, DEGW), lambda i: (0, i, 0)),
        ],
        out_specs=[
            pl.BlockSpec((_B, DH), lambda i: (i, 0)),
            pl.BlockSpec((_B, DH), lambda i: (i, 0)),
            pl.BlockSpec((_B, 1), lambda i: (i, 0)),
        ],
        out_shape=[
            jax.ShapeDtypeStruct((N, DH), jnp.float32),
            jax.ShapeDtypeStruct((N, DH), jnp.float32),
            jax.ShapeDtypeStruct((N, 1), jnp.float32),
        ],
    )(h1, degp)


def _stage2_body(alo_ref, ahi_ref, hlo_ref, hhi_ref, dinv_ref, b_ref, w_ref,
                 lo_ref, hi_ref):
    dinv = dinv_ref[...]
    a_lo = alo_ref[0] + alo_ref[1] + hlo_ref[...]
    a_hi = ahi_ref[0] + ahi_ref[1] + hhi_ref[...]
    a = jnp.concatenate([a_lo, a_hi], axis=1)
    z = jnp.maximum(dinv * a + b_ref[...], 0.0)
    hs = jnp.dot(z, w_ref[...], preferred_element_type=jnp.float32) * dinv
    lo_ref[...] = hs[:, :DH]
    hi_ref[...] = hs[:, DH:]


def _tc_stage2(alo, ahi, hlo, hhi, dinv, b, W):
    return pl.pallas_call(
        _stage2_body,
        grid=(_GRID,),
        in_specs=[
            pl.BlockSpec((NC, _B, DH), lambda i: (0, i, 0)),
            pl.BlockSpec((NC, _B, DH), lambda i: (0, i, 0)),
            pl.BlockSpec((_B, DH), lambda i: (i, 0)),
            pl.BlockSpec((_B, DH), lambda i: (i, 0)),
            pl.BlockSpec((_B, 1), lambda i: (i, 0)),
            pl.BlockSpec((1, D), lambda i: (0, 0)),
            pl.BlockSpec((D, D), lambda i: (0, 0)),
        ],
        out_specs=[
            pl.BlockSpec((_B, DH), lambda i: (i, 0)),
            pl.BlockSpec((_B, DH), lambda i: (i, 0)),
        ],
        out_shape=[
            jax.ShapeDtypeStruct((N, DH), jnp.float32),
            jax.ShapeDtypeStruct((N, DH), jnp.float32),
        ],
    )(alo, ahi, hlo, hhi, dinv, b, W)


def _stage3_body(alo_ref, ahi_ref, hlo_ref, hhi_ref, dinv_ref, b_ref,
                 wfc_ref, bfc_ref, out_ref):
    a_lo = alo_ref[0] + alo_ref[1] + hlo_ref[...]
    a_hi = ahi_ref[0] + ahi_ref[1] + hhi_ref[...]
    a = jnp.concatenate([a_lo, a_hi], axis=1)
    z = jnp.maximum(dinv_ref[...] * a + b_ref[...], 0.0)
    out_ref[...] = (
        jnp.dot(z, wfc_ref[...], preferred_element_type=jnp.float32) + bfc_ref[...]
    )


def _tc_stage3(alo, ahi, hlo, hhi, dinv, b, Wfc8, bfc8):
    return pl.pallas_call(
        _stage3_body,
        grid=(_GRID,),
        in_specs=[
            pl.BlockSpec((NC, _B, DH), lambda i: (0, i, 0)),
            pl.BlockSpec((NC, _B, DH), lambda i: (0, i, 0)),
            pl.BlockSpec((_B, DH), lambda i: (i, 0)),
            pl.BlockSpec((_B, DH), lambda i: (i, 0)),
            pl.BlockSpec((_B, 1), lambda i: (i, 0)),
            pl.BlockSpec((1, D), lambda i: (0, 0)),
            pl.BlockSpec((D, DEGW), lambda i: (0, 0)),
            pl.BlockSpec((1, DEGW), lambda i: (0, 0)),
        ],
        out_specs=pl.BlockSpec((_B, DEGW), lambda i: (i, 0)),
        out_shape=jax.ShapeDtypeStruct((N, DEGW), jnp.float32),
    )(alo, ahi, hlo, hhi, dinv, b, Wfc8, bfc8)


def kernel(x, edge_index, W1, b1, W2, b2, Wfc, bfc):
    src = edge_index[0]
    dst = edge_index[1]
    pad = EPAD - E
    ar = jnp.arange(pad, dtype=jnp.int32)
    src3 = jnp.concatenate([src, (ar * 997) % N]).reshape(NW, K, C)
    dst3 = jnp.concatenate([dst, N + (ar % (NPAD - N))]).reshape(NW, K, C)

    ones = jnp.ones((C, DEGW), jnp.float32)
    zeros = jnp.zeros((ROWS_PER_TILE, DEGW), jnp.float32)

    h1 = _tc_stage1a(x, W1)
    degp = _sc_degree(dst3, ones, zeros)
    h1lo, h1hi, dinv = _tc_stage1(h1, degp)
    a1lo, a1hi = _sc_scatter(h1lo, h1hi, src3, dst3)
    h2lo, h2hi = _tc_stage2(a1lo, a1hi, h1lo, h1hi, dinv, b1.reshape(1, D), W2)
    a2lo, a2hi = _sc_scatter(h2lo, h2hi, src3, dst3)
    Wfc8 = jnp.pad(Wfc, ((0, 0), (0, DEGW - 2)))
    bfc8 = jnp.pad(bfc, (0, DEGW - 2)).reshape(1, DEGW)
    out8 = _tc_stage3(a2lo, a2hi, h2lo, h2hi, dinv, b2.reshape(1, D), Wfc8, bfc8)
    return out8[:, :2]
